# baseline (speedup 1.0000x reference)
import functools
import math

import numpy as np

import jax
import jax.numpy as jnp
from jax import lax
from jax.experimental import pallas as pl
from jax.experimental.pallas import tpu as pltpu
from jax.experimental.pallas import tpu_sc as plsc

_SEL = 513
_SENTINEL = 50.0


def _np_threefry2x32(k1, k2, x0, x1):
    def rotl(x, d):
        return ((x << np.uint32(d)) | (x >> np.uint32(32 - d))).astype(np.uint32)

    ks = [np.uint32(k1), np.uint32(k2),
          np.uint32(np.uint32(k1) ^ np.uint32(k2) ^ np.uint32(0x1BD11BDA))]
    rotations = [[13, 15, 26, 6], [17, 29, 16, 24]]
    x0 = (x0 + ks[0]).astype(np.uint32)
    x1 = (x1 + ks[1]).astype(np.uint32)
    for i in range(5):
        for r in rotations[i % 2]:
            x0 = (x0 + x1).astype(np.uint32)
            x1 = rotl(x1, r)
            x1 = x1 ^ x0
        x0 = (x0 + ks[(i + 1) % 3]).astype(np.uint32)
        x1 = (x1 + ks[(i + 2) % 3] + np.uint32(i + 1)).astype(np.uint32)
    return x0, x1


def _np_uniform_key42(shape):
    size = int(np.prod(shape))
    idx = np.arange(size, dtype=np.uint64)
    c1 = (idx >> np.uint64(32)).astype(np.uint32)
    c2 = (idx & np.uint64(0xFFFFFFFF)).astype(np.uint32)
    b1, b2 = _np_threefry2x32(0, 42, c1, c2)
    bits = (b1 ^ b2).astype(np.uint32)
    fb = (bits >> np.uint32(9)) | np.uint32(0x3F800000)
    f = fb.view(np.float32) - np.float32(1.0)
    mn = np.float32(np.finfo(np.float32).tiny)
    u = np.maximum(mn, f * (np.float32(1.0) - mn) + mn).astype(np.float32)
    return u.reshape(shape)


def _z_kernel(q_ref, k_ref, u_ref, z_ref):
    kk = k_ref[0]
    a = jax.lax.dot_general(
        q_ref[0], kk, (((1,), (0,)), ((), ())),
        preferred_element_type=jnp.float32,
    )
    a = a / math.sqrt(kk.shape[0])
    col = jax.lax.broadcasted_iota(jnp.int32, a.shape, 1)
    is0 = col == 0
    am = jnp.where(is0, -jnp.inf, a)
    m = jnp.max(am)
    e = jnp.where(is0, 0.0, jnp.exp(am - m))
    p = e / jnp.sum(e)
    g = -jnp.log(-jnp.log(u_ref[0]))
    z = jnp.log(p + 1e-20) + g
    z_ref[0] = jnp.where(is0, _SENTINEL, z)


def _sc_select(B, S):
    NC, NS, L = 2, 16, 16
    NW = NC * NS
    ROWS_PER_W = B // NW
    NCH = S // L
    mesh = plsc.VectorSubcoreMesh(core_axis_name="c", subcore_axis_name="s",
                                  num_cores=NC, num_subcores=NS)

    M31 = jnp.int32(0x7FFFFFFF)
    SIGN = jnp.int32(-2147483648)

    def scalar(x):
        return lax.reduce_max(x, (0,))

    @functools.partial(
        pl.kernel, mesh=mesh,
        out_type=jax.ShapeDtypeStruct((B, S), jnp.int32),
        compiler_params=pltpu.CompilerParams(needs_layout_passes=False),
        scratch_types=[
            pltpu.VMEM((S,), jnp.float32),
            pltpu.VMEM((S,), jnp.int32),
            pltpu.VMEM((256,), jnp.int32),
        ],
    )
    def sel(z_hbm, out_hbm, zrow, mrow, hist):
        wid = lax.axis_index("s") * NC + lax.axis_index("c")

        def ukey_of(x):
            s = lax.bitcast_convert_type(x, jnp.int32)
            key = jnp.where(s < 0, jnp.int32(-1) - (s & M31), s)
            return key ^ SIGN

        for r in range(ROWS_PER_W):
            row = wid * ROWS_PER_W + r
            pltpu.sync_copy(z_hbm.at[row], zrow)

            prefix = jnp.int32(0)
            remaining = jnp.int32(_SEL)
            ones = jnp.ones((L,), jnp.int32)
            for p in range(4):
                shift = 24 - 8 * p
                for c in range(16):
                    hist[pl.ds(c * L, L)] = jnp.zeros((L,), jnp.int32)

                pfx = prefix

                def hbody(c, carry):
                    u = ukey_of(zrow[pl.ds(c * L, L)])
                    dig = lax.shift_right_logical(u, shift) & jnp.int32(0xFF)
                    if p == 0:
                        plsc.addupdate_scatter(hist, [dig], ones)
                    else:
                        hi = lax.shift_right_logical(u, shift + 8)
                        plsc.addupdate_scatter(hist, [dig], ones,
                                               mask=hi == pfx)
                    return carry

                jax.lax.fori_loop(0, NCH, hbody, jnp.int32(0))

                def sbody(i, st):
                    carry, found, dstar, rem = st
                    c = 15 - i
                    h = hist[pl.ds(c * L, L)]
                    sfx = lax.rev(plsc.cumsum(lax.rev(h, (0,))), (0,))
                    ge = sfx + carry
                    cond = ge >= rem
                    npos = scalar(plsc.all_reduce_population_count(cond))
                    here = (1 - found) * jnp.where(npos > 0, 1, 0)
                    istar = jnp.maximum(npos - 1, 0)
                    lane = lax.broadcasted_iota(jnp.int32, (L,), 0)
                    sel_ge = jnp.max(jnp.where(lane == istar, ge,
                                               jnp.int32(-2147483647)))
                    sel_h = jnp.max(jnp.where(lane == istar, h,
                                              jnp.int32(-2147483647)))
                    dstar = jnp.where(here == 1, c * L + istar, dstar)
                    rem = jnp.where(here == 1, rem - (sel_ge - sel_h), rem)
                    found = found | here
                    carry = carry + scalar(plsc.cumsum(h))
                    return carry, found, dstar, rem

                _, _, dstar, remaining = jax.lax.fori_loop(
                    0, 16, sbody,
                    (jnp.int32(0), jnp.int32(0), jnp.int32(0), remaining))
                prefix = (prefix << 8) | dstar

            ustar = prefix
            kstar = ustar ^ SIGN

            def mbody(c, ties_seen):
                u = ukey_of(zrow[pl.ds(c * L, L)])
                key = u ^ SIGN
                gt = (key > kstar).astype(jnp.int32)
                tie = (key == kstar).astype(jnp.int32)
                pc = plsc.cumsum(tie)
                keep = tie * jnp.where((ties_seen + pc) <= remaining, 1, 0)
                mrow[pl.ds(c * L, L)] = gt | keep
                return ties_seen + scalar(pc)

            jax.lax.fori_loop(0, NCH, mbody, jnp.int32(0))
            pltpu.sync_copy(mrow, out_hbm.at[row])

    return sel


def kernel(q, k):
    B, S, D = q.shape
    gp_np = np.full((B, 1, S), 0.5, np.float32)
    gp_np[:, 0, 1:] = _np_uniform_key42((B, S - 1))
    gp = jnp.asarray(gp_np)
    q0 = q[:, :1, :]
    kt = jnp.swapaxes(k, 1, 2)
    z = pl.pallas_call(
        _z_kernel,
        grid=(B,),
        in_specs=[
            pl.BlockSpec((1, 1, D), lambda b: (b, 0, 0)),
            pl.BlockSpec((1, D, S), lambda b: (b, 0, 0)),
            pl.BlockSpec((1, 1, S), lambda b: (b, 0, 0)),
        ],
        out_specs=pl.BlockSpec((1, 1, S), lambda b: (b, 0, 0)),
        out_shape=jax.ShapeDtypeStruct((B, 1, S), jnp.float32),
    )(q0, kt, gp).reshape(B, S)
    m = _sc_select(B, S)(z)
    return m.astype(bool)


# speedup vs baseline: 1.6805x; 1.6805x over previous
import math

import numpy as np

import jax
import jax.numpy as jnp
from jax.experimental import pallas as pl
from jax.experimental.pallas import tpu as pltpu

_SEL = 513
_SENTINEL = 50.0


def _np_threefry2x32(k1, k2, x0, x1):
    def rotl(x, d):
        return ((x << np.uint32(d)) | (x >> np.uint32(32 - d))).astype(np.uint32)

    ks = [np.uint32(k1), np.uint32(k2),
          np.uint32(np.uint32(k1) ^ np.uint32(k2) ^ np.uint32(0x1BD11BDA))]
    rotations = [[13, 15, 26, 6], [17, 29, 16, 24]]
    x0 = (x0 + ks[0]).astype(np.uint32)
    x1 = (x1 + ks[1]).astype(np.uint32)
    for i in range(5):
        for r in rotations[i % 2]:
            x0 = (x0 + x1).astype(np.uint32)
            x1 = rotl(x1, r)
            x1 = x1 ^ x0
        x0 = (x0 + ks[(i + 1) % 3]).astype(np.uint32)
        x1 = (x1 + ks[(i + 2) % 3] + np.uint32(i + 1)).astype(np.uint32)
    return x0, x1


def _np_uniform_key42(shape):
    size = int(np.prod(shape))
    idx = np.arange(size, dtype=np.uint64)
    c1 = (idx >> np.uint64(32)).astype(np.uint32)
    c2 = (idx & np.uint64(0xFFFFFFFF)).astype(np.uint32)
    b1, b2 = _np_threefry2x32(0, 42, c1, c2)
    bits = (b1 ^ b2).astype(np.uint32)
    fb = (bits >> np.uint32(9)) | np.uint32(0x3F800000)
    f = fb.view(np.float32) - np.float32(1.0)
    mn = np.float32(np.finfo(np.float32).tiny)
    u = np.maximum(mn, f * (np.float32(1.0) - mn) + mn).astype(np.float32)
    return u.reshape(shape)


def _make_kernel(B, S, D):
    def body(q_ref, k_ref, u_ref, o_ref, z_scr):
        b = pl.program_id(0)

        @pl.when(b < B)
        def _compute_z():
            kk = k_ref[0]
            a = jax.lax.dot_general(
                q_ref[0], kk, (((1,), (0,)), ((), ())),
                preferred_element_type=jnp.float32,
            )
            a = a / math.sqrt(D)
            col = jax.lax.broadcasted_iota(jnp.int32, a.shape, 1)
            is0 = col == 0
            am = jnp.where(is0, -jnp.inf, a)
            m = jnp.max(am)
            e = jnp.where(is0, 0.0, jnp.exp(am - m))
            p = e / jnp.sum(e)
            g = -jnp.log(-jnp.log(u_ref[0]))
            z = jnp.log(p + 1e-20) + g
            z_scr[pl.ds(b, 1), :] = jnp.where(is0, _SENTINEL, z)

        @pl.when(b == B)
        def _select():
            z = z_scr[...]
            s = jax.lax.bitcast_convert_type(z, jnp.int32)
            key = jnp.where(s < 0, jnp.int32(-1) - (s & jnp.int32(0x7FFFFFFF)), s)
            col = jax.lax.broadcasted_iota(jnp.int32, key.shape, 1)

            def count_ge(t):
                return jnp.sum((key >= t).astype(jnp.int32), axis=1, keepdims=True)

            lo = jnp.min(key, axis=1, keepdims=True)
            hi = jnp.max(key, axis=1, keepdims=True)

            def bisect(_, lohi):
                lo, hi = lohi
                mid = (lo >> 1) + (hi >> 1) + (lo & hi & 1)
                pred = count_ge(mid) >= _SEL
                return jnp.where(pred, mid, lo), jnp.where(pred, hi, mid)

            v, _ = jax.lax.fori_loop(0, 32, bisect, (lo, hi))
            cnt_gt = count_ge(v + 1)
            need = _SEL - cnt_gt
            tie = (key == v).astype(jnp.int32)

            def cutsearch(_, clochi):
                clo, chi = clochi
                mid = (clo + chi) >> 1
                cnt = jnp.sum(jnp.where(col <= mid, tie, 0), axis=1, keepdims=True)
                pred = cnt >= need
                return jnp.where(pred, clo, mid), jnp.where(pred, mid, chi)

            clo = jnp.full_like(v, -1)
            chi = jnp.full_like(v, S - 1)
            _, cut = jax.lax.fori_loop(0, 14, cutsearch, (clo, chi))
            mask = (key > v) | ((key == v) & (col <= cut))
            o_ref[...] = mask.astype(jnp.int8)

    return body


def kernel(q, k):
    B, S, D = q.shape
    up_np = np.full((B, 1, S), 0.5, np.float32)
    up_np[:, 0, 1:] = _np_uniform_key42((B, S - 1))
    up = jnp.asarray(up_np)
    q0 = q[:, :1, :]
    kt = jnp.swapaxes(k, 1, 2)
    last = B - 1
    m = pl.pallas_call(
        _make_kernel(B, S, D),
        grid=(B + 1,),
        in_specs=[
            pl.BlockSpec((1, 1, D), lambda b: (jnp.minimum(b, last), 0, 0)),
            pl.BlockSpec((1, D, S), lambda b: (jnp.minimum(b, last), 0, 0)),
            pl.BlockSpec((1, 1, S), lambda b: (jnp.minimum(b, last), 0, 0)),
        ],
        out_specs=pl.BlockSpec((B, S), lambda b: (0, 0)),
        out_shape=jax.ShapeDtypeStruct((B, S), jnp.int8),
        scratch_shapes=[pltpu.VMEM((B, S), jnp.float32)],
    )(q0, kt, up)
    return m.astype(bool)
